# baseline (speedup 1.0000x reference)
import functools

import jax
import jax.numpy as jnp
from jax import lax
from jax.experimental import pallas as pl
from jax.experimental.pallas import tpu as pltpu
from jax.experimental.pallas import tpu_sc as plsc

_P = 1024
_E = 768
_H = _E // 2
_M = 16
_L = 16
_NC, _NS = 2, 16
_NW = _NC * _NS
_PPW = _P // _NW
_NV = _H // _L


def _sc_body(idx_hbm, x_hbm, sp_hbm, im_hbm, out_hbm,
             idx_v, sp_v, row_v,
             in0, in1, ou0, ou1,
             sem_r, si0, si1, so0, so1):
    nbatch = x_hbm.shape[0] // _P
    wid = lax.axis_index("s") * _NC + lax.axis_index("c")
    base = wid * _PPW

    pltpu.sync_copy(idx_hbm, idx_v)
    pltpu.async_copy(im_hbm.at[idx_v], row_v, sem_r).wait()
    pltpu.sync_copy(sp_hbm.at[pl.ds(base, _PPW)], sp_v)
    rvals = [row_v[0, pl.ds(_L * j, _L)] for j in range(_NV)]

    ins = (in0, in1)
    ous = (ou0, ou1)
    sis = (si0, si1)
    sos = (so0, so1)

    def rows(b):
        return pl.ds(b * _P + base, _PPW)

    pltpu.async_copy(x_hbm.at[rows(0)], in0, si0)
    pltpu.async_copy(x_hbm.at[rows(1)], in1, si1)

    def add_chunk(ibuf, obuf):
        def body(p, _):
            for j in range(_NV):
                s = pl.ds(_L * j, _L)
                obuf[p, s] = ibuf[p, s] + sp_v[p, s]
            for j in range(_NV):
                s = pl.ds(_H + _L * j, _L)
                obuf[p, s] = ibuf[p, s] + rvals[j]
            return 0
        lax.fori_loop(0, _PPW, body, 0)

    def step(half, _):
        b0 = half * 2
        for i in range(2):
            b = b0 + i

            @pl.when(b >= 2)
            def _wait_out():
                pltpu.make_async_copy(ous[i], out_hbm.at[rows(b)], sos[i]).wait()

            pltpu.make_async_copy(x_hbm.at[rows(b)], ins[i], sis[i]).wait()
            add_chunk(ins[i], ous[i])
            pltpu.async_copy(ous[i], out_hbm.at[rows(b)], sos[i])

            @pl.when(b + 2 < nbatch)
            def _next_in():
                pltpu.async_copy(x_hbm.at[rows(b + 2)], ins[i], sis[i])
        return 0

    lax.fori_loop(0, nbatch // 2, step, 0)
    for i in range(2):
        pltpu.make_async_copy(ous[i], out_hbm.at[rows(nbatch - 2 + i)], sos[i]).wait()


def kernel(x, image_idx, spatial_pos_embed, image_pos_embed):
    B, P, E = x.shape
    idx = jnp.asarray(image_idx, jnp.int32).reshape(1)
    mesh = plsc.VectorSubcoreMesh(core_axis_name="c", subcore_axis_name="s")
    out = pl.kernel(
        _sc_body,
        mesh=mesh,
        out_type=jax.ShapeDtypeStruct((B * P, E), jnp.float32),
        scratch_types=[
            pltpu.VMEM((1,), jnp.int32),
            pltpu.VMEM((_PPW, _H), jnp.float32),
            pltpu.VMEM((1, _H), jnp.float32),
            pltpu.VMEM((_PPW, _E), jnp.float32),
            pltpu.VMEM((_PPW, _E), jnp.float32),
            pltpu.VMEM((_PPW, _E), jnp.float32),
            pltpu.VMEM((_PPW, _E), jnp.float32),
            pltpu.SemaphoreType.DMA,
            pltpu.SemaphoreType.DMA,
            pltpu.SemaphoreType.DMA,
            pltpu.SemaphoreType.DMA,
            pltpu.SemaphoreType.DMA,
        ],
    )(idx, x.reshape(B * P, E), spatial_pos_embed.reshape(P, _H),
      image_pos_embed.reshape(_M, _H))
    return out.reshape(B, P, E)


# speedup vs baseline: 2.3246x; 2.3246x over previous
import jax
import jax.numpy as jnp
from jax import lax
from jax.experimental import pallas as pl
from jax.experimental.pallas import tpu as pltpu
from jax.experimental.pallas import tpu_sc as plsc

_P = 1024
_E = 768
_H = _E // 2
_M = 16
_NC = 2


def _row_lookup_sc(idx_hbm, im_hbm, row_hbm, idx_v, row_v, sem):
    wid = lax.axis_index("s") * _NC + lax.axis_index("c")

    @pl.when(wid == 0)
    def _():
        pltpu.sync_copy(idx_hbm, idx_v)
        pltpu.async_copy(im_hbm.at[idx_v], row_v, sem).wait()
        pltpu.sync_copy(row_v, row_hbm)


def _image_row(idx, image2d):
    mesh = plsc.VectorSubcoreMesh(core_axis_name="c", subcore_axis_name="s")
    return pl.kernel(
        _row_lookup_sc,
        mesh=mesh,
        out_type=jax.ShapeDtypeStruct((1, _H), jnp.float32),
        scratch_types=[
            pltpu.VMEM((1,), jnp.int32),
            pltpu.VMEM((1, _H), jnp.float32),
            pltpu.SemaphoreType.DMA,
        ],
    )(idx, image2d)


def _add_body(x_ref, sp_ref, row_ref, o_ref):
    h = sp_ref.shape[-1]
    o_ref[:, :, :h] = x_ref[:, :, :h] + sp_ref[:]
    o_ref[:, :, h:] = x_ref[:, :, h:] + row_ref[0][None, None, :]


def kernel(x, image_idx, spatial_pos_embed, image_pos_embed):
    B, P, E = x.shape
    idx = jnp.asarray(image_idx, jnp.int32).reshape(1)
    row = _image_row(idx, image_pos_embed.reshape(_M, _H))
    bb = 4
    return pl.pallas_call(
        _add_body,
        grid=(B // bb,),
        in_specs=[
            pl.BlockSpec((bb, P, E), lambda b: (b, 0, 0)),
            pl.BlockSpec((1, P, _H), lambda b: (0, 0, 0)),
            pl.BlockSpec((1, _H), lambda b: (0, 0)),
        ],
        out_specs=pl.BlockSpec((bb, P, E), lambda b: (b, 0, 0)),
        out_shape=jax.ShapeDtypeStruct((B, P, E), x.dtype),
        compiler_params=pltpu.CompilerParams(
            dimension_semantics=("arbitrary",),
        ),
    )(x, spatial_pos_embed, row)


# speedup vs baseline: 2.3493x; 1.0106x over previous
import jax
import jax.numpy as jnp
from jax import lax
from jax.experimental import pallas as pl
from jax.experimental.pallas import tpu as pltpu
from jax.experimental.pallas import tpu_sc as plsc

_P = 1024
_E = 768
_H = _E // 2
_M = 16
_NC = 2


def _row_lookup_sc(idx_hbm, im_hbm, row_hbm, idx_s):
    @pl.when(lax.axis_index("c") == 0)
    def _():
        pltpu.sync_copy(idx_hbm, idx_s)
        pltpu.sync_copy(im_hbm.at[pl.ds(idx_s[0], 1)], row_hbm)


def _image_row(idx, image2d):
    mesh = plsc.ScalarSubcoreMesh(axis_name="c", num_cores=_NC)
    return pl.kernel(
        _row_lookup_sc,
        mesh=mesh,
        out_type=jax.ShapeDtypeStruct((1, _H), jnp.float32),
        scratch_types=[
            pltpu.SMEM((1,), jnp.int32),
        ],
    )(idx, image2d)


def _add_body(x_ref, sp_ref, row_ref, o_ref):
    h = sp_ref.shape[-1]
    o_ref[:, :, :h] = x_ref[:, :, :h] + sp_ref[:]
    o_ref[:, :, h:] = x_ref[:, :, h:] + row_ref[0][None, None, :]


def kernel(x, image_idx, spatial_pos_embed, image_pos_embed):
    B, P, E = x.shape
    idx = jnp.asarray(image_idx, jnp.int32).reshape(1)
    row = _image_row(idx, image_pos_embed.reshape(_M, _H))
    bb = 4
    return pl.pallas_call(
        _add_body,
        grid=(B // bb,),
        in_specs=[
            pl.BlockSpec((bb, P, E), lambda b: (b, 0, 0)),
            pl.BlockSpec((1, P, _H), lambda b: (0, 0, 0)),
            pl.BlockSpec((1, _H), lambda b: (0, 0)),
        ],
        out_specs=pl.BlockSpec((bb, P, E), lambda b: (b, 0, 0)),
        out_shape=jax.ShapeDtypeStruct((B, P, E), x.dtype),
        compiler_params=pltpu.CompilerParams(
            dimension_semantics=("arbitrary",),
        ),
    )(x, spatial_pos_embed, row)


# speedup vs baseline: 2.3741x; 1.0106x over previous
import jax
import jax.numpy as jnp
from jax import lax
from jax.experimental import pallas as pl
from jax.experimental.pallas import tpu as pltpu
from jax.experimental.pallas import tpu_sc as plsc

_P = 1024
_E = 768
_H = _E // 2
_M = 16
_NC = 2


def _row_lookup_sc(idx_hbm, im_hbm, row_hbm, idx_s):
    @pl.when(lax.axis_index("c") == 0)
    def _():
        pltpu.sync_copy(idx_hbm, idx_s)
        pltpu.sync_copy(im_hbm.at[pl.ds(idx_s[0], 1)], row_hbm)


def _image_row(idx, image2d):
    mesh = plsc.ScalarSubcoreMesh(axis_name="c", num_cores=1)
    return pl.kernel(
        _row_lookup_sc,
        mesh=mesh,
        out_type=jax.ShapeDtypeStruct((1, _H), jnp.float32),
        scratch_types=[
            pltpu.SMEM((1,), jnp.int32),
        ],
    )(idx, image2d)


def _add_body(x_ref, sp_ref, row_ref, o_ref):
    h = sp_ref.shape[-1]
    o_ref[:, :, :h] = x_ref[:, :, :h] + sp_ref[:]
    o_ref[:, :, h:] = x_ref[:, :, h:] + row_ref[0][None, None, :]


def kernel(x, image_idx, spatial_pos_embed, image_pos_embed):
    B, P, E = x.shape
    idx = jnp.asarray(image_idx, jnp.int32).reshape(1)
    row = _image_row(idx, image_pos_embed.reshape(_M, _H))
    bb = 4
    return pl.pallas_call(
        _add_body,
        grid=(B // bb,),
        in_specs=[
            pl.BlockSpec((bb, P, E), lambda b: (b, 0, 0)),
            pl.BlockSpec((1, P, _H), lambda b: (0, 0, 0)),
            pl.BlockSpec((1, _H), lambda b: (0, 0)),
        ],
        out_specs=pl.BlockSpec((bb, P, E), lambda b: (b, 0, 0)),
        out_shape=jax.ShapeDtypeStruct((B, P, E), x.dtype),
        compiler_params=pltpu.CompilerParams(
            dimension_semantics=("arbitrary",),
        ),
    )(x, spatial_pos_embed, row)
